# baseline (speedup 1.0000x reference)
import functools

import jax
import jax.numpy as jnp
from jax import lax
from jax.experimental import pallas as pl
from jax.experimental.pallas import tpu as pltpu
from jax.experimental.pallas import tpu_sc as plsc

N = 10000
E = 320000
DF = 128
DH = 128
DC = 64

NC = 2
NS = 16
NW = NC * NS

NPAD = 10240
RT = NPAD // NS

CH = 80
EPT = E // NW
NCHUNK = EPT // CH

R = 256
GRID = NPAD // R

_mesh = plsc.VectorSubcoreMesh(core_axis_name="c", subcore_axis_name="s")


@functools.partial(
    pl.kernel,
    out_type=jax.ShapeDtypeStruct((NC, NPAD), jnp.float32),
    mesh=_mesh,
    scratch_types=[
        pltpu.VMEM((EPT,), jnp.int32),
        pltpu.VMEM((CH,), jnp.float32),
        pltpu.VMEM((RT,), jnp.float32),
        pltpu.VMEM_SHARED((NPAD,), jnp.float32),
    ],
)
def _deg_kernel(dst_hbm, out_hbm, dst_v, ones_v, zeros_v, acc_sh):
    c = lax.axis_index("c")
    s = lax.axis_index("s")
    wid = c * NS + s
    for k in range(RT // 16):
        zeros_v[pl.ds(k * 16, 16)] = jnp.zeros((16,), jnp.float32)
    for k in range(CH // 16):
        ones_v[pl.ds(k * 16, 16)] = jnp.ones((16,), jnp.float32)
    pltpu.sync_copy(zeros_v, acc_sh.at[pl.ds(s * RT, RT)])
    pltpu.sync_copy(dst_hbm.at[pl.ds(wid * EPT, EPT)], dst_v)
    plsc.subcore_barrier()

    def body(i, carry):
        pltpu.sync_copy(ones_v, acc_sh.at[dst_v.at[pl.ds(i * CH, CH)]],
                        add=True)
        return carry

    lax.fori_loop(0, NCHUNK, body, 0)
    plsc.subcore_barrier()
    pltpu.sync_copy(acc_sh.at[pl.ds(s * RT, RT)], out_hbm.at[c, pl.ds(s * RT, RT)])


def _make_agg(d_feat):
    @functools.partial(
        pl.kernel,
        out_type=jax.ShapeDtypeStruct((NC, NPAD, d_feat), jnp.float32),
        mesh=_mesh,
        scratch_types=[
            pltpu.VMEM((EPT,), jnp.int32),
            pltpu.VMEM((EPT,), jnp.int32),
            pltpu.VMEM((CH, d_feat), jnp.float32),
            pltpu.VMEM((CH, d_feat), jnp.float32),
            pltpu.VMEM_SHARED((NPAD, d_feat), jnp.float32),
            pltpu.SemaphoreType.DMA,
            pltpu.SemaphoreType.DMA,
            pltpu.SemaphoreType.DMA,
            pltpu.SemaphoreType.DMA,
        ],
    )
    def _agg_kernel(h_hbm, src_hbm, dst_hbm, zeros_hbm, out_hbm,
                    src_v, dst_v, rows_a, rows_b, acc_sh,
                    gsem_a, gsem_b, ssem_a, ssem_b):
        c = lax.axis_index("c")
        s = lax.axis_index("s")
        wid = c * NS + s
        pltpu.sync_copy(zeros_hbm.at[pl.ds(s * RT, RT)],
                        acc_sh.at[pl.ds(s * RT, RT)])
        pltpu.sync_copy(src_hbm.at[pl.ds(wid * EPT, EPT)], src_v)
        pltpu.sync_copy(dst_hbm.at[pl.ds(wid * EPT, EPT)], dst_v)
        plsc.subcore_barrier()

        def _gather(k, buf, sem):
            return pltpu.async_copy(
                h_hbm.at[src_v.at[pl.ds(k * CH, CH)]], buf, sem)

        def _scatter(k, buf, sem):
            return pltpu.async_copy(
                buf, acc_sh.at[dst_v.at[pl.ds(k * CH, CH)]], sem, add=True)

        def _wait_gather(k, buf, sem):
            pltpu.make_async_copy(
                h_hbm.at[src_v.at[pl.ds(k * CH, CH)]], buf, sem).wait()

        def _wait_scatter(k, buf, sem):
            pltpu.make_async_copy(
                buf, acc_sh.at[dst_v.at[pl.ds(k * CH, CH)]], sem).wait()

        _gather(0, rows_a, gsem_a)
        _gather(1, rows_b, gsem_b)

        def body(j, carry):
            a = 2 * j
            b = a + 1
            _wait_gather(a, rows_a, gsem_a)
            _scatter(a, rows_a, ssem_a)
            _wait_gather(b, rows_b, gsem_b)
            _scatter(b, rows_b, ssem_b)
            _wait_scatter(a, rows_a, ssem_a)

            @pl.when(j < NCHUNK // 2 - 1)
            def _start_next_a():
                _gather(a + 2, rows_a, gsem_a)

            _wait_scatter(b, rows_b, ssem_b)

            @pl.when(j < NCHUNK // 2 - 1)
            def _start_next_b():
                _gather(b + 2, rows_b, gsem_b)

            return carry

        lax.fori_loop(0, NCHUNK // 2, body, 0)
        if NCHUNK % 2:
            _gather(NCHUNK - 1, rows_a, gsem_a).wait()
            _scatter(NCHUNK - 1, rows_a, ssem_a)
            _wait_scatter(NCHUNK - 1, rows_a, ssem_a)
        plsc.subcore_barrier()
        pltpu.sync_copy(acc_sh.at[pl.ds(s * RT, RT)],
                        out_hbm.at[c, pl.ds(s * RT, RT)])

    return _agg_kernel


_agg128 = _make_agg(DH)


def _dinv_col(degp_ref):
    dsum = degp_ref[0, :] + degp_ref[1, :] + 1.0
    dl = lax.rsqrt(dsum)
    rows = lax.broadcasted_iota(jnp.int32, (R, R), 0)
    cols = lax.broadcasted_iota(jnp.int32, (R, R), 1)
    diag = jnp.where(rows == cols, dl[None, :], 0.0)
    return jnp.sum(diag, axis=1, keepdims=True)


def _mm1_body(x_ref, w_ref, h_ref):
    h_ref[...] = jnp.dot(x_ref[...], w_ref[...],
                         preferred_element_type=jnp.float32)


def _scale_body(h_ref, degp_ref, hs_ref, dinv_ref):
    col = _dinv_col(degp_ref)
    hs_ref[...] = h_ref[...] * col
    dinv_ref[...] = col


def _mm2_body(aggp_ref, h1_ref, dinv_ref, b1_ref, w2_ref, h2_ref):
    agg = aggp_ref[0] + aggp_ref[1]
    z = dinv_ref[...] * (agg + h1_ref[...]) + b1_ref[...]
    z = jnp.maximum(z, 0.0)
    h2 = jnp.dot(z, w2_ref[...], preferred_element_type=jnp.float32)
    h2_ref[...] = jnp.concatenate(
        [h2 * dinv_ref[...], jnp.zeros((R, DH - DC), jnp.float32)], axis=1)


def _out_body(aggp_ref, h2_ref, dinv_ref, b2_ref, out_ref):
    acc = aggp_ref[0] + aggp_ref[1] + h2_ref[...]
    z = dinv_ref[...] * acc[:, :DC] + b2_ref[...]
    m = jnp.max(z, axis=1, keepdims=True)
    lse = jnp.log(jnp.sum(jnp.exp(z - m), axis=1, keepdims=True))
    out_ref[...] = z - m - lse


def _tc_mm1(x, W1):
    return pl.pallas_call(
        _mm1_body,
        grid=(GRID,),
        in_specs=[
            pl.BlockSpec((R, DF), lambda i: (i, 0)),
            pl.BlockSpec((DF, DH), lambda i: (0, 0)),
        ],
        out_specs=pl.BlockSpec((R, DH), lambda i: (i, 0)),
        out_shape=jax.ShapeDtypeStruct((NPAD, DH), jnp.float32),
    )(x, W1)


def _tc_scale(h1, degp):
    return pl.pallas_call(
        _scale_body,
        grid=(GRID,),
        in_specs=[
            pl.BlockSpec((R, DH), lambda i: (i, 0)),
            pl.BlockSpec((NC, R), lambda i: (0, i)),
        ],
        out_specs=[
            pl.BlockSpec((R, DH), lambda i: (i, 0)),
            pl.BlockSpec((R, 1), lambda i: (i, 0)),
        ],
        out_shape=[
            jax.ShapeDtypeStruct((NPAD, DH), jnp.float32),
            jax.ShapeDtypeStruct((NPAD, 1), jnp.float32),
        ],
    )(h1, degp)


def _tc_layer2(agg1, h1p, dinvc, b1, W2):
    return pl.pallas_call(
        _mm2_body,
        grid=(GRID,),
        in_specs=[
            pl.BlockSpec((NC, R, DH), lambda i: (0, i, 0)),
            pl.BlockSpec((R, DH), lambda i: (i, 0)),
            pl.BlockSpec((R, 1), lambda i: (i, 0)),
            pl.BlockSpec((1, DH), lambda i: (0, 0)),
            pl.BlockSpec((DH, DC), lambda i: (0, 0)),
        ],
        out_specs=pl.BlockSpec((R, DH), lambda i: (i, 0)),
        out_shape=jax.ShapeDtypeStruct((NPAD, DH), jnp.float32),
    )(agg1, h1p, dinvc, b1, W2)


def _tc_out(agg2, h2p, dinvc, b2):
    return pl.pallas_call(
        _out_body,
        grid=(GRID,),
        in_specs=[
            pl.BlockSpec((NC, R, DH), lambda i: (0, i, 0)),
            pl.BlockSpec((R, DH), lambda i: (i, 0)),
            pl.BlockSpec((R, 1), lambda i: (i, 0)),
            pl.BlockSpec((1, DC), lambda i: (0, 0)),
        ],
        out_specs=pl.BlockSpec((R, DC), lambda i: (i, 0)),
        out_shape=jax.ShapeDtypeStruct((N, DC), jnp.float32),
    )(agg2, h2p, dinvc, b2)


def kernel(features, edges, W1, b1, W2, b2):
    src = edges[0].astype(jnp.int32)
    dst = edges[1].astype(jnp.int32)

    degp = _deg_kernel(dst)
    h1 = _tc_mm1(features, W1)
    h1p, dinvc = _tc_scale(h1, degp)
    agg1 = _agg128(h1p, src, dst, jnp.zeros((NPAD, DH), jnp.float32))
    h2p = _tc_layer2(agg1, h1p, dinvc, b1.reshape(1, DH), W2)
    agg2 = _agg128(h2p, src, dst, jnp.zeros((NPAD, DH), jnp.float32))
    return _tc_out(agg2, h2p, dinvc, b2.reshape(1, DC))


# speedup vs baseline: 1.1822x; 1.1822x over previous
import functools

import jax
import jax.numpy as jnp
from jax import lax
from jax.experimental import pallas as pl
from jax.experimental.pallas import tpu as pltpu
from jax.experimental.pallas import tpu_sc as plsc

N = 10000
E = 320000
DF = 128
DH = 128
DC = 64

NC = 2
NS = 16
NW = NC * NS

NPAD = 10240
RT = NPAD // NS

CH = 80
EPT = E // NW
NCHUNK = EPT // CH

R = 256
GRID = NPAD // R

_mesh = plsc.VectorSubcoreMesh(core_axis_name="c", subcore_axis_name="s")


@functools.partial(
    pl.kernel,
    out_type=jax.ShapeDtypeStruct((NC, NPAD), jnp.float32),
    mesh=_mesh,
    scratch_types=[
        pltpu.VMEM((EPT,), jnp.int32),
        pltpu.VMEM((CH,), jnp.float32),
        pltpu.VMEM((RT,), jnp.float32),
        pltpu.VMEM_SHARED((NPAD,), jnp.float32),
    ],
)
def _deg_kernel(dst_hbm, out_hbm, dst_v, ones_v, zeros_v, acc_sh):
    c = lax.axis_index("c")
    s = lax.axis_index("s")
    wid = c * NS + s
    for k in range(RT // 16):
        zeros_v[pl.ds(k * 16, 16)] = jnp.zeros((16,), jnp.float32)
    for k in range(CH // 16):
        ones_v[pl.ds(k * 16, 16)] = jnp.ones((16,), jnp.float32)
    pltpu.sync_copy(zeros_v, acc_sh.at[pl.ds(s * RT, RT)])
    pltpu.sync_copy(dst_hbm.at[pl.ds(wid * EPT, EPT)], dst_v)
    plsc.subcore_barrier()

    def body(i, carry):
        pltpu.sync_copy(ones_v, acc_sh.at[dst_v.at[pl.ds(i * CH, CH)]],
                        add=True)
        return carry

    lax.fori_loop(0, NCHUNK, body, 0)
    plsc.subcore_barrier()
    pltpu.sync_copy(acc_sh.at[pl.ds(s * RT, RT)], out_hbm.at[c, pl.ds(s * RT, RT)])


def _make_agg(d_feat):
    @functools.partial(
        pl.kernel,
        out_type=jax.ShapeDtypeStruct((NC, NPAD, d_feat), jnp.float32),
        mesh=_mesh,
        scratch_types=[
            pltpu.VMEM((EPT,), jnp.int32),
            pltpu.VMEM((EPT,), jnp.int32),
            pltpu.VMEM((CH, d_feat), jnp.float32),
            pltpu.VMEM((CH, d_feat), jnp.float32),
            pltpu.VMEM_SHARED((NPAD, d_feat), jnp.float32),
            pltpu.SemaphoreType.DMA,
            pltpu.SemaphoreType.DMA,
        ],
    )
    def _agg_kernel(h_hbm, src_hbm, dst_hbm, zeros_hbm, out_hbm,
                    src_v, dst_v, rows_a, rows_b, acc_sh, gsem_a, gsem_b):
        c = lax.axis_index("c")
        s = lax.axis_index("s")
        wid = c * NS + s
        pltpu.sync_copy(zeros_hbm.at[pl.ds(s * RT, RT)],
                        acc_sh.at[pl.ds(s * RT, RT)])
        pltpu.sync_copy(src_hbm.at[pl.ds(wid * EPT, EPT)], src_v)
        pltpu.sync_copy(dst_hbm.at[pl.ds(wid * EPT, EPT)], dst_v)
        plsc.subcore_barrier()

        def _gather(k, buf, sem):
            return pltpu.async_copy(
                h_hbm.at[src_v.at[pl.ds(k * CH, CH)]], buf, sem)

        def _scatter(k, buf):
            pltpu.sync_copy(buf, acc_sh.at[dst_v.at[pl.ds(k * CH, CH)]],
                            add=True)

        def _wait_gather(k, buf, sem):
            pltpu.make_async_copy(
                h_hbm.at[src_v.at[pl.ds(k * CH, CH)]], buf, sem).wait()

        _gather(0, rows_a, gsem_a)

        def body(j, carry):
            a = 2 * j
            b = a + 1
            _gather(b, rows_b, gsem_b)
            _wait_gather(a, rows_a, gsem_a)
            _scatter(a, rows_a)

            @pl.when(j < NCHUNK // 2 - 1)
            def _start_next_a():
                _gather(a + 2, rows_a, gsem_a)

            _wait_gather(b, rows_b, gsem_b)
            _scatter(b, rows_b)
            return carry

        lax.fori_loop(0, NCHUNK // 2, body, 0)
        if NCHUNK % 2:
            _gather(NCHUNK - 1, rows_a, gsem_a).wait()
            _scatter(NCHUNK - 1, rows_a)
        plsc.subcore_barrier()
        pltpu.sync_copy(acc_sh.at[pl.ds(s * RT, RT)],
                        out_hbm.at[c, pl.ds(s * RT, RT)])

    return _agg_kernel


_agg128 = _make_agg(DH)


def _dinv_col(degp_ref):
    dsum = degp_ref[0, :] + degp_ref[1, :] + 1.0
    dl = lax.rsqrt(dsum)
    rows = lax.broadcasted_iota(jnp.int32, (R, R), 0)
    cols = lax.broadcasted_iota(jnp.int32, (R, R), 1)
    diag = jnp.where(rows == cols, dl[None, :], 0.0)
    return jnp.sum(diag, axis=1, keepdims=True)


def _mm1_body(x_ref, w_ref, h_ref):
    h_ref[...] = jnp.dot(x_ref[...], w_ref[...],
                         preferred_element_type=jnp.float32)


def _scale_body(h_ref, degp_ref, hs_ref, dinv_ref):
    col = _dinv_col(degp_ref)
    hs_ref[...] = h_ref[...] * col
    dinv_ref[...] = col


def _mm2_body(aggp_ref, h1_ref, dinv_ref, b1_ref, w2_ref, h2_ref):
    agg = aggp_ref[0] + aggp_ref[1]
    z = dinv_ref[...] * (agg + h1_ref[...]) + b1_ref[...]
    z = jnp.maximum(z, 0.0)
    h2 = jnp.dot(z, w2_ref[...], preferred_element_type=jnp.float32)
    h2_ref[...] = jnp.concatenate(
        [h2 * dinv_ref[...], jnp.zeros((R, DH - DC), jnp.float32)], axis=1)


def _out_body(aggp_ref, h2_ref, dinv_ref, b2_ref, out_ref):
    acc = aggp_ref[0] + aggp_ref[1] + h2_ref[...]
    z = dinv_ref[...] * acc[:, :DC] + b2_ref[...]
    m = jnp.max(z, axis=1, keepdims=True)
    lse = jnp.log(jnp.sum(jnp.exp(z - m), axis=1, keepdims=True))
    out_ref[...] = z - m - lse


def _tc_mm1(x, W1):
    return pl.pallas_call(
        _mm1_body,
        grid=(GRID,),
        in_specs=[
            pl.BlockSpec((R, DF), lambda i: (i, 0)),
            pl.BlockSpec((DF, DH), lambda i: (0, 0)),
        ],
        out_specs=pl.BlockSpec((R, DH), lambda i: (i, 0)),
        out_shape=jax.ShapeDtypeStruct((NPAD, DH), jnp.float32),
    )(x, W1)


def _tc_scale(h1, degp):
    return pl.pallas_call(
        _scale_body,
        grid=(GRID,),
        in_specs=[
            pl.BlockSpec((R, DH), lambda i: (i, 0)),
            pl.BlockSpec((NC, R), lambda i: (0, i)),
        ],
        out_specs=[
            pl.BlockSpec((R, DH), lambda i: (i, 0)),
            pl.BlockSpec((R, 1), lambda i: (i, 0)),
        ],
        out_shape=[
            jax.ShapeDtypeStruct((NPAD, DH), jnp.float32),
            jax.ShapeDtypeStruct((NPAD, 1), jnp.float32),
        ],
    )(h1, degp)


def _tc_layer2(agg1, h1p, dinvc, b1, W2):
    return pl.pallas_call(
        _mm2_body,
        grid=(GRID,),
        in_specs=[
            pl.BlockSpec((NC, R, DH), lambda i: (0, i, 0)),
            pl.BlockSpec((R, DH), lambda i: (i, 0)),
            pl.BlockSpec((R, 1), lambda i: (i, 0)),
            pl.BlockSpec((1, DH), lambda i: (0, 0)),
            pl.BlockSpec((DH, DC), lambda i: (0, 0)),
        ],
        out_specs=pl.BlockSpec((R, DH), lambda i: (i, 0)),
        out_shape=jax.ShapeDtypeStruct((NPAD, DH), jnp.float32),
    )(agg1, h1p, dinvc, b1, W2)


def _tc_out(agg2, h2p, dinvc, b2):
    return pl.pallas_call(
        _out_body,
        grid=(GRID,),
        in_specs=[
            pl.BlockSpec((NC, R, DH), lambda i: (0, i, 0)),
            pl.BlockSpec((R, DH), lambda i: (i, 0)),
            pl.BlockSpec((R, 1), lambda i: (i, 0)),
            pl.BlockSpec((1, DC), lambda i: (0, 0)),
        ],
        out_specs=pl.BlockSpec((R, DC), lambda i: (i, 0)),
        out_shape=jax.ShapeDtypeStruct((N, DC), jnp.float32),
    )(agg2, h2p, dinvc, b2)


def kernel(features, edges, W1, b1, W2, b2):
    src = edges[0].astype(jnp.int32)
    dst = edges[1].astype(jnp.int32)

    degp = _deg_kernel(dst)
    h1 = _tc_mm1(features, W1)
    h1p, dinvc = _tc_scale(h1, degp)
    agg1 = _agg128(h1p, src, dst, jnp.zeros((NPAD, DH), jnp.float32))
    h2p = _tc_layer2(agg1, h1p, dinvc, b1.reshape(1, DH), W2)
    agg2 = _agg128(h2p, src, dst, jnp.zeros((NPAD, DH), jnp.float32))
    return _tc_out(agg2, h2p, dinvc, b2.reshape(1, DC))


# speedup vs baseline: 1.3750x; 1.1631x over previous
import functools

import jax
import jax.numpy as jnp
from jax import lax
from jax.experimental import pallas as pl
from jax.experimental.pallas import tpu as pltpu
from jax.experimental.pallas import tpu_sc as plsc

N = 10000
E = 320000
DF = 128
DH = 128
DC = 64

NC = 2
NS = 16
NW = NC * NS

NPAD = 10240
RT = NPAD // NS

CH = 80
EPT = E // NW
NCHUNK = EPT // CH

R = 256
GRID = NPAD // R

_mesh = plsc.VectorSubcoreMesh(core_axis_name="c", subcore_axis_name="s")


@functools.partial(
    pl.kernel,
    out_type=jax.ShapeDtypeStruct((NC, NPAD), jnp.float32),
    mesh=_mesh,
    scratch_types=[
        pltpu.VMEM((EPT,), jnp.int32),
        pltpu.VMEM((CH,), jnp.float32),
        pltpu.VMEM((RT,), jnp.float32),
        pltpu.VMEM_SHARED((NPAD,), jnp.float32),
    ],
)
def _deg_kernel(dst_hbm, out_hbm, dst_v, ones_v, zeros_v, acc_sh):
    c = lax.axis_index("c")
    s = lax.axis_index("s")
    wid = c * NS + s
    for k in range(RT // 16):
        zeros_v[pl.ds(k * 16, 16)] = jnp.zeros((16,), jnp.float32)
    for k in range(CH // 16):
        ones_v[pl.ds(k * 16, 16)] = jnp.ones((16,), jnp.float32)
    pltpu.sync_copy(zeros_v, acc_sh.at[pl.ds(s * RT, RT)])
    pltpu.sync_copy(dst_hbm.at[pl.ds(wid * EPT, EPT)], dst_v)
    plsc.subcore_barrier()

    def body(i, carry):
        pltpu.sync_copy(ones_v, acc_sh.at[dst_v.at[pl.ds(i * CH, CH)]],
                        add=True)
        return carry

    lax.fori_loop(0, NCHUNK, body, 0)
    plsc.subcore_barrier()
    pltpu.sync_copy(acc_sh.at[pl.ds(s * RT, RT)], out_hbm.at[c, pl.ds(s * RT, RT)])


NBUF = 4
NIDX = 8


def _make_agg(d_feat):
    @functools.partial(
        pl.kernel,
        out_type=jax.ShapeDtypeStruct((NC, NPAD, d_feat), jnp.float32),
        mesh=_mesh,
        scratch_types=(
            [pltpu.VMEM((CH, d_feat), jnp.float32) for _ in range(NBUF)]
            + [pltpu.VMEM((CH,), jnp.int32) for _ in range(NIDX)]
            + [pltpu.VMEM((CH,), jnp.int32) for _ in range(NIDX)]
            + [pltpu.VMEM_SHARED((NPAD, d_feat), jnp.float32)]
            + [pltpu.SemaphoreType.DMA for _ in range(NBUF + NIDX)]
        ),
    )
    def _agg_kernel(h_hbm, src_hbm, dst_hbm, zeros_hbm, out_hbm, *rest):
        rows = rest[:NBUF]
        srci = rest[NBUF:NBUF + NIDX]
        dsti = rest[NBUF + NIDX:NBUF + 2 * NIDX]
        acc_sh = rest[NBUF + 2 * NIDX]
        rsem = rest[NBUF + 2 * NIDX + 1:NBUF + 2 * NIDX + 1 + NBUF]
        isem = rest[NBUF + 2 * NIDX + 1 + NBUF:]
        c = lax.axis_index("c")
        s = lax.axis_index("s")
        base = (c * NS + s) * EPT

        def _idx_copy(k, q):
            pltpu.async_copy(src_hbm.at[pl.ds(base + k * CH, CH)],
                             srci[q], isem[q])
            pltpu.async_copy(dst_hbm.at[pl.ds(base + k * CH, CH)],
                             dsti[q], isem[q])

        def _wait_idx(k, q):
            pltpu.make_async_copy(src_hbm.at[pl.ds(base + k * CH, CH)],
                                  srci[q], isem[q]).wait()
            pltpu.make_async_copy(dst_hbm.at[pl.ds(base + k * CH, CH)],
                                  dsti[q], isem[q]).wait()

        def _gather(b, q):
            pltpu.async_copy(h_hbm.at[srci[q]], rows[b], rsem[b])

        def _wait_gather(b, q):
            pltpu.make_async_copy(h_hbm.at[srci[q]], rows[b],
                                  rsem[b]).wait()

        def _scatter(b, q):
            pltpu.sync_copy(rows[b], acc_sh.at[dsti[q]], add=True)

        for q in range(NIDX):
            _idx_copy(q, q)
        pltpu.sync_copy(zeros_hbm.at[pl.ds(s * RT, RT)],
                        acc_sh.at[pl.ds(s * RT, RT)])
        plsc.subcore_barrier()
        for b in range(NBUF):
            _wait_idx(b, b)
            _gather(b, b)

        NFULL = (NCHUNK // NIDX) * NIDX

        def body(j, carry):
            for u in range(NIDX):
                k = j * NIDX + u
                _wait_gather(u % NBUF, u)
                _scatter(u % NBUF, u)
                nq = (u + NBUF) % NIDX
                _wait_idx(k + NBUF, nq)
                _gather(u % NBUF, nq)
                nk8 = k + NIDX

                @pl.when(nk8 < NCHUNK)
                def _refill_idx():
                    _idx_copy(nk8, u)

            return carry

        lax.fori_loop(0, NFULL // NIDX, body, 0)
        for k in range(NFULL, NCHUNK):
            _wait_gather(k % NBUF, k % NIDX)
            _scatter(k % NBUF, k % NIDX)
            nk = k + NBUF
            if nk < NCHUNK:
                _wait_idx(nk, nk % NIDX)
                _gather(nk % NBUF, nk % NIDX)
        plsc.subcore_barrier()
        pltpu.sync_copy(acc_sh.at[pl.ds(s * RT, RT)],
                        out_hbm.at[c, pl.ds(s * RT, RT)])

    return _agg_kernel


_agg128 = _make_agg(DH)


def _dinv_col(degp_ref):
    dsum = degp_ref[0, :] + degp_ref[1, :] + 1.0
    dl = lax.rsqrt(dsum)
    rows = lax.broadcasted_iota(jnp.int32, (R, R), 0)
    cols = lax.broadcasted_iota(jnp.int32, (R, R), 1)
    diag = jnp.where(rows == cols, dl[None, :], 0.0)
    return jnp.sum(diag, axis=1, keepdims=True)


def _mm1_body(x_ref, w_ref, h_ref):
    h_ref[...] = jnp.dot(x_ref[...], w_ref[...],
                         preferred_element_type=jnp.float32)


def _scale_body(h_ref, degp_ref, hs_ref, dinv_ref):
    col = _dinv_col(degp_ref)
    hs_ref[...] = h_ref[...] * col
    dinv_ref[...] = col


def _mm2_body(aggp_ref, h1_ref, dinv_ref, b1_ref, w2_ref, h2_ref):
    agg = aggp_ref[0] + aggp_ref[1]
    z = dinv_ref[...] * (agg + h1_ref[...]) + b1_ref[...]
    z = jnp.maximum(z, 0.0)
    h2 = jnp.dot(z, w2_ref[...], preferred_element_type=jnp.float32)
    h2_ref[...] = jnp.concatenate(
        [h2 * dinv_ref[...], jnp.zeros((R, DH - DC), jnp.float32)], axis=1)


def _out_body(aggp_ref, h2_ref, dinv_ref, b2_ref, out_ref):
    acc = aggp_ref[0] + aggp_ref[1] + h2_ref[...]
    z = dinv_ref[...] * acc[:, :DC] + b2_ref[...]
    m = jnp.max(z, axis=1, keepdims=True)
    lse = jnp.log(jnp.sum(jnp.exp(z - m), axis=1, keepdims=True))
    out_ref[...] = z - m - lse


def _tc_mm1(x, W1):
    return pl.pallas_call(
        _mm1_body,
        grid=(GRID,),
        in_specs=[
            pl.BlockSpec((R, DF), lambda i: (i, 0)),
            pl.BlockSpec((DF, DH), lambda i: (0, 0)),
        ],
        out_specs=pl.BlockSpec((R, DH), lambda i: (i, 0)),
        out_shape=jax.ShapeDtypeStruct((NPAD, DH), jnp.float32),
    )(x, W1)


def _tc_scale(h1, degp):
    return pl.pallas_call(
        _scale_body,
        grid=(GRID,),
        in_specs=[
            pl.BlockSpec((R, DH), lambda i: (i, 0)),
            pl.BlockSpec((NC, R), lambda i: (0, i)),
        ],
        out_specs=[
            pl.BlockSpec((R, DH), lambda i: (i, 0)),
            pl.BlockSpec((R, 1), lambda i: (i, 0)),
        ],
        out_shape=[
            jax.ShapeDtypeStruct((NPAD, DH), jnp.float32),
            jax.ShapeDtypeStruct((NPAD, 1), jnp.float32),
        ],
    )(h1, degp)


def _tc_layer2(agg1, h1p, dinvc, b1, W2):
    return pl.pallas_call(
        _mm2_body,
        grid=(GRID,),
        in_specs=[
            pl.BlockSpec((NC, R, DH), lambda i: (0, i, 0)),
            pl.BlockSpec((R, DH), lambda i: (i, 0)),
            pl.BlockSpec((R, 1), lambda i: (i, 0)),
            pl.BlockSpec((1, DH), lambda i: (0, 0)),
            pl.BlockSpec((DH, DC), lambda i: (0, 0)),
        ],
        out_specs=pl.BlockSpec((R, DH), lambda i: (i, 0)),
        out_shape=jax.ShapeDtypeStruct((NPAD, DH), jnp.float32),
    )(agg1, h1p, dinvc, b1, W2)


def _tc_out(agg2, h2p, dinvc, b2):
    return pl.pallas_call(
        _out_body,
        grid=(GRID,),
        in_specs=[
            pl.BlockSpec((NC, R, DH), lambda i: (0, i, 0)),
            pl.BlockSpec((R, DH), lambda i: (i, 0)),
            pl.BlockSpec((R, 1), lambda i: (i, 0)),
            pl.BlockSpec((1, DC), lambda i: (0, 0)),
        ],
        out_specs=pl.BlockSpec((R, DC), lambda i: (i, 0)),
        out_shape=jax.ShapeDtypeStruct((N, DC), jnp.float32),
    )(agg2, h2p, dinvc, b2)


def kernel(features, edges, W1, b1, W2, b2):
    src = edges[0].astype(jnp.int32)
    dst = edges[1].astype(jnp.int32)

    degp = _deg_kernel(dst)
    h1 = _tc_mm1(features, W1)
    h1p, dinvc = _tc_scale(h1, degp)
    agg1 = _agg128(h1p, src, dst, jnp.zeros((NPAD, DH), jnp.float32))
    h2p = _tc_layer2(agg1, h1p, dinvc, b1.reshape(1, DH), W2)
    agg2 = _agg128(h2p, src, dst, jnp.zeros((NPAD, DH), jnp.float32))
    return _tc_out(agg2, h2p, dinvc, b2.reshape(1, DC))


# speedup vs baseline: 1.6971x; 1.2342x over previous
import functools

import jax
import jax.numpy as jnp
from jax import lax
from jax.experimental import pallas as pl
from jax.experimental.pallas import tpu as pltpu
from jax.experimental.pallas import tpu_sc as plsc

N = 10000
E = 320000
DF = 128
DH = 128
DC = 64

NC = 2
NS = 16
NW = NC * NS

NPAD = 10240
RT = NPAD // NS

CH = 80
EPT = E // NW
NCHUNK = EPT // CH

R = 2048
GRID = NPAD // R

_mesh = plsc.VectorSubcoreMesh(core_axis_name="c", subcore_axis_name="s")


@functools.partial(
    pl.kernel,
    out_type=jax.ShapeDtypeStruct((NC, NPAD), jnp.float32),
    mesh=_mesh,
    scratch_types=[
        pltpu.VMEM((EPT,), jnp.int32),
        pltpu.VMEM((CH,), jnp.float32),
        pltpu.VMEM((RT,), jnp.float32),
        pltpu.VMEM_SHARED((NPAD,), jnp.float32),
    ],
)
def _deg_kernel(dst_hbm, out_hbm, dst_v, ones_v, zeros_v, acc_sh):
    c = lax.axis_index("c")
    s = lax.axis_index("s")
    wid = c * NS + s
    for k in range(RT // 16):
        zeros_v[pl.ds(k * 16, 16)] = jnp.zeros((16,), jnp.float32)
    for k in range(CH // 16):
        ones_v[pl.ds(k * 16, 16)] = jnp.ones((16,), jnp.float32)
    pltpu.sync_copy(zeros_v, acc_sh.at[pl.ds(s * RT, RT)])
    pltpu.sync_copy(dst_hbm.at[pl.ds(wid * EPT, EPT)], dst_v)
    plsc.subcore_barrier()

    def body(i, carry):
        pltpu.sync_copy(ones_v, acc_sh.at[dst_v.at[pl.ds(i * CH, CH)]],
                        add=True)
        return carry

    lax.fori_loop(0, NCHUNK, body, 0)
    plsc.subcore_barrier()
    pltpu.sync_copy(acc_sh.at[pl.ds(s * RT, RT)], out_hbm.at[c, pl.ds(s * RT, RT)])


NBUF = 4
NIDX = 8


def _make_agg(d_feat):
    @functools.partial(
        pl.kernel,
        out_type=jax.ShapeDtypeStruct((NC, NPAD, d_feat), jnp.float32),
        mesh=_mesh,
        scratch_types=(
            [pltpu.VMEM((CH, d_feat), jnp.float32) for _ in range(NBUF)]
            + [pltpu.VMEM((CH,), jnp.int32) for _ in range(NIDX)]
            + [pltpu.VMEM((CH,), jnp.int32) for _ in range(NIDX)]
            + [pltpu.VMEM_SHARED((NPAD, d_feat), jnp.float32)]
            + [pltpu.SemaphoreType.DMA for _ in range(NBUF + NIDX)]
        ),
    )
    def _agg_kernel(h_hbm, src_hbm, dst_hbm, zeros_hbm, out_hbm, *rest):
        rows = rest[:NBUF]
        srci = rest[NBUF:NBUF + NIDX]
        dsti = rest[NBUF + NIDX:NBUF + 2 * NIDX]
        acc_sh = rest[NBUF + 2 * NIDX]
        rsem = rest[NBUF + 2 * NIDX + 1:NBUF + 2 * NIDX + 1 + NBUF]
        isem = rest[NBUF + 2 * NIDX + 1 + NBUF:]
        c = lax.axis_index("c")
        s = lax.axis_index("s")
        base = (c * NS + s) * EPT

        def _idx_copy(k, q):
            pltpu.async_copy(src_hbm.at[pl.ds(base + k * CH, CH)],
                             srci[q], isem[q])
            pltpu.async_copy(dst_hbm.at[pl.ds(base + k * CH, CH)],
                             dsti[q], isem[q])

        def _wait_idx(k, q):
            pltpu.make_async_copy(src_hbm.at[pl.ds(base + k * CH, CH)],
                                  srci[q], isem[q]).wait()
            pltpu.make_async_copy(dst_hbm.at[pl.ds(base + k * CH, CH)],
                                  dsti[q], isem[q]).wait()

        def _gather(b, q):
            pltpu.async_copy(h_hbm.at[srci[q]], rows[b], rsem[b])

        def _wait_gather(b, q):
            pltpu.make_async_copy(h_hbm.at[srci[q]], rows[b],
                                  rsem[b]).wait()

        def _scatter(b, q):
            pltpu.sync_copy(rows[b], acc_sh.at[dsti[q]], add=True)

        for q in range(NIDX):
            _idx_copy(q, q)
        pltpu.sync_copy(zeros_hbm.at[pl.ds(s * RT, RT)],
                        acc_sh.at[pl.ds(s * RT, RT)])
        plsc.subcore_barrier()
        for b in range(NBUF):
            _wait_idx(b, b)
            _gather(b, b)

        NFULL = (NCHUNK // NIDX) * NIDX

        def body(j, carry):
            for u in range(NIDX):
                k = j * NIDX + u
                _wait_gather(u % NBUF, u)
                _scatter(u % NBUF, u)
                nq = (u + NBUF) % NIDX
                _wait_idx(k + NBUF, nq)
                _gather(u % NBUF, nq)
                nk8 = k + NIDX

                @pl.when(nk8 < NCHUNK)
                def _refill_idx():
                    _idx_copy(nk8, u)

            return carry

        lax.fori_loop(0, NFULL // NIDX, body, 0)
        for k in range(NFULL, NCHUNK):
            _wait_gather(k % NBUF, k % NIDX)
            _scatter(k % NBUF, k % NIDX)
            nk = k + NBUF
            if nk < NCHUNK:
                _wait_idx(nk, nk % NIDX)
                _gather(nk % NBUF, nk % NIDX)
        plsc.subcore_barrier()
        pltpu.sync_copy(acc_sh.at[pl.ds(s * RT, RT)],
                        out_hbm.at[c, pl.ds(s * RT, RT)])

    return _agg_kernel


_agg128 = _make_agg(DH)


_TR = 256


def _dinv_col(degp_ref):
    dsum = degp_ref[0, :] + degp_ref[1, :] + 1.0
    dl = lax.rsqrt(dsum)
    rows = lax.broadcasted_iota(jnp.int32, (_TR, _TR), 0)
    cols = lax.broadcasted_iota(jnp.int32, (_TR, _TR), 1)
    eye = (rows == cols)
    cols_out = []
    for t in range(R // _TR):
        diag = jnp.where(eye, dl[None, t * _TR:(t + 1) * _TR], 0.0)
        cols_out.append(jnp.sum(diag, axis=1, keepdims=True))
    return jnp.concatenate(cols_out, axis=0)


def _mm1_body(x_ref, w_ref, h_ref):
    h_ref[...] = jnp.dot(x_ref[...], w_ref[...],
                         preferred_element_type=jnp.float32)


def _scale_body(h_ref, degp_ref, hs_ref, dinv_ref):
    col = _dinv_col(degp_ref)
    hs_ref[...] = h_ref[...] * col
    dinv_ref[...] = col


def _mm2_body(aggp_ref, h1_ref, dinv_ref, b1_ref, w2_ref, h2_ref):
    agg = aggp_ref[0] + aggp_ref[1]
    z = dinv_ref[...] * (agg + h1_ref[...]) + b1_ref[...]
    z = jnp.maximum(z, 0.0)
    h2 = jnp.dot(z, w2_ref[...], preferred_element_type=jnp.float32)
    h2_ref[...] = jnp.concatenate(
        [h2 * dinv_ref[...], jnp.zeros((R, DH - DC), jnp.float32)], axis=1)


def _out_body(aggp_ref, h2_ref, dinv_ref, b2_ref, out_ref):
    acc = aggp_ref[0] + aggp_ref[1] + h2_ref[...]
    z = dinv_ref[...] * acc[:, :DC] + b2_ref[...]
    m = jnp.max(z, axis=1, keepdims=True)
    lse = jnp.log(jnp.sum(jnp.exp(z - m), axis=1, keepdims=True))
    out_ref[...] = z - m - lse


def _tc_mm1(x, W1):
    return pl.pallas_call(
        _mm1_body,
        grid=(GRID,),
        in_specs=[
            pl.BlockSpec((R, DF), lambda i: (i, 0)),
            pl.BlockSpec((DF, DH), lambda i: (0, 0)),
        ],
        out_specs=pl.BlockSpec((R, DH), lambda i: (i, 0)),
        out_shape=jax.ShapeDtypeStruct((NPAD, DH), jnp.float32),
    )(x, W1)


def _tc_scale(h1, degp):
    return pl.pallas_call(
        _scale_body,
        grid=(GRID,),
        in_specs=[
            pl.BlockSpec((R, DH), lambda i: (i, 0)),
            pl.BlockSpec((NC, R), lambda i: (0, i)),
        ],
        out_specs=[
            pl.BlockSpec((R, DH), lambda i: (i, 0)),
            pl.BlockSpec((R, 1), lambda i: (i, 0)),
        ],
        out_shape=[
            jax.ShapeDtypeStruct((NPAD, DH), jnp.float32),
            jax.ShapeDtypeStruct((NPAD, 1), jnp.float32),
        ],
    )(h1, degp)


def _tc_layer2(agg1, h1p, dinvc, b1, W2):
    return pl.pallas_call(
        _mm2_body,
        grid=(GRID,),
        in_specs=[
            pl.BlockSpec((NC, R, DH), lambda i: (0, i, 0)),
            pl.BlockSpec((R, DH), lambda i: (i, 0)),
            pl.BlockSpec((R, 1), lambda i: (i, 0)),
            pl.BlockSpec((1, DH), lambda i: (0, 0)),
            pl.BlockSpec((DH, DC), lambda i: (0, 0)),
        ],
        out_specs=pl.BlockSpec((R, DH), lambda i: (i, 0)),
        out_shape=jax.ShapeDtypeStruct((NPAD, DH), jnp.float32),
    )(agg1, h1p, dinvc, b1, W2)


def _tc_out(agg2, h2p, dinvc, b2):
    return pl.pallas_call(
        _out_body,
        grid=(GRID,),
        in_specs=[
            pl.BlockSpec((NC, R, DH), lambda i: (0, i, 0)),
            pl.BlockSpec((R, DH), lambda i: (i, 0)),
            pl.BlockSpec((R, 1), lambda i: (i, 0)),
            pl.BlockSpec((1, DC), lambda i: (0, 0)),
        ],
        out_specs=pl.BlockSpec((R, DC), lambda i: (i, 0)),
        out_shape=jax.ShapeDtypeStruct((N, DC), jnp.float32),
    )(agg2, h2p, dinvc, b2)


def kernel(features, edges, W1, b1, W2, b2):
    src = edges[0].astype(jnp.int32)
    dst = edges[1].astype(jnp.int32)

    degp = _deg_kernel(dst)
    h1 = _tc_mm1(features, W1)
    h1p, dinvc = _tc_scale(h1, degp)
    agg1 = _agg128(h1p, src, dst, jnp.zeros((NPAD, DH), jnp.float32))
    h2p = _tc_layer2(agg1, h1p, dinvc, b1.reshape(1, DH), W2)
    agg2 = _agg128(h2p, src, dst, jnp.zeros((NPAD, DH), jnp.float32))
    return _tc_out(agg2, h2p, dinvc, b2.reshape(1, DC))


# speedup vs baseline: 1.7814x; 1.0497x over previous
import functools

import jax
import jax.numpy as jnp
from jax import lax
from jax.experimental import pallas as pl
from jax.experimental.pallas import tpu as pltpu
from jax.experimental.pallas import tpu_sc as plsc

N = 10000
E = 320000
DF = 128
DH = 128
DC = 64

NC = 2
NS = 16
NW = NC * NS

NPAD = 10240
RT = NPAD // NS

CH = 80
EPT = E // NW
NCHUNK = EPT // CH

R = 2048
GRID = NPAD // R

_mesh = plsc.VectorSubcoreMesh(core_axis_name="c", subcore_axis_name="s")


@functools.partial(
    pl.kernel,
    out_type=jax.ShapeDtypeStruct((NC, NPAD), jnp.float32),
    mesh=_mesh,
    scratch_types=[
        pltpu.VMEM((EPT,), jnp.int32),
        pltpu.VMEM((CH,), jnp.float32),
        pltpu.VMEM((RT,), jnp.float32),
        pltpu.VMEM_SHARED((NPAD,), jnp.float32),
    ],
)
def _deg_kernel(dst_hbm, out_hbm, dst_v, ones_v, zeros_v, acc_sh):
    c = lax.axis_index("c")
    s = lax.axis_index("s")
    wid = c * NS + s
    for k in range(RT // 16):
        zeros_v[pl.ds(k * 16, 16)] = jnp.zeros((16,), jnp.float32)
    for k in range(CH // 16):
        ones_v[pl.ds(k * 16, 16)] = jnp.ones((16,), jnp.float32)
    pltpu.sync_copy(zeros_v, acc_sh.at[pl.ds(s * RT, RT)])
    pltpu.sync_copy(dst_hbm.at[pl.ds(wid * EPT, EPT)], dst_v)
    plsc.subcore_barrier()

    def body(i, carry):
        pltpu.sync_copy(ones_v, acc_sh.at[dst_v.at[pl.ds(i * CH, CH)]],
                        add=True)
        return carry

    lax.fori_loop(0, NCHUNK, body, 0)
    plsc.subcore_barrier()
    pltpu.sync_copy(acc_sh.at[pl.ds(s * RT, RT)], out_hbm.at[c, pl.ds(s * RT, RT)])


NBUF = 4
NIDX = 8


def _make_agg(d_feat):
    @functools.partial(
        pl.kernel,
        out_type=jax.ShapeDtypeStruct((NC, NPAD, d_feat), jnp.float32),
        mesh=_mesh,
        scratch_types=(
            [pltpu.VMEM((CH, d_feat), jnp.float32) for _ in range(NBUF)]
            + [pltpu.VMEM((CH,), jnp.int32) for _ in range(NIDX)]
            + [pltpu.VMEM((CH,), jnp.int32) for _ in range(NIDX)]
            + [pltpu.VMEM_SHARED((NPAD, d_feat), jnp.float32)]
            + [pltpu.SemaphoreType.DMA for _ in range(NBUF + NIDX)]
        ),
    )
    def _agg_kernel(h_hbm, src_hbm, dst_hbm, zeros_hbm, out_hbm, *rest):
        rows = rest[:NBUF]
        srci = rest[NBUF:NBUF + NIDX]
        dsti = rest[NBUF + NIDX:NBUF + 2 * NIDX]
        acc_sh = rest[NBUF + 2 * NIDX]
        rsem = rest[NBUF + 2 * NIDX + 1:NBUF + 2 * NIDX + 1 + NBUF]
        isem = rest[NBUF + 2 * NIDX + 1 + NBUF:]
        c = lax.axis_index("c")
        s = lax.axis_index("s")
        base = (c * NS + s) * EPT

        def _idx_copy(k, q):
            pltpu.async_copy(src_hbm.at[pl.ds(base + k * CH, CH)],
                             srci[q], isem[q])
            pltpu.async_copy(dst_hbm.at[pl.ds(base + k * CH, CH)],
                             dsti[q], isem[q])

        def _wait_idx(k, q):
            pltpu.make_async_copy(src_hbm.at[pl.ds(base + k * CH, CH)],
                                  srci[q], isem[q]).wait()
            pltpu.make_async_copy(dst_hbm.at[pl.ds(base + k * CH, CH)],
                                  dsti[q], isem[q]).wait()

        def _gather(b, q):
            pltpu.async_copy(h_hbm.at[srci[q]], rows[b], rsem[b])

        def _wait_gather(b, q):
            pltpu.make_async_copy(h_hbm.at[srci[q]], rows[b],
                                  rsem[b]).wait()

        def _scatter(b, q):
            pltpu.sync_copy(rows[b], acc_sh.at[dsti[q]], add=True)

        for q in range(NIDX):
            _idx_copy(q, q)
        pltpu.sync_copy(zeros_hbm, acc_sh.at[pl.ds(s * RT, RT)])
        plsc.subcore_barrier()
        for b in range(NBUF):
            _wait_idx(b, b)
            _gather(b, b)

        NFULL = (NCHUNK // NIDX) * NIDX

        def body(j, carry):
            for u in range(NIDX):
                k = j * NIDX + u
                _wait_gather(u % NBUF, u)
                _scatter(u % NBUF, u)
                nq = (u + NBUF) % NIDX
                _wait_idx(k + NBUF, nq)
                _gather(u % NBUF, nq)
                nk8 = k + NIDX

                @pl.when(nk8 < NCHUNK)
                def _refill_idx():
                    _idx_copy(nk8, u)

            return carry

        lax.fori_loop(0, NFULL // NIDX, body, 0)
        for k in range(NFULL, NCHUNK):
            _wait_gather(k % NBUF, k % NIDX)
            _scatter(k % NBUF, k % NIDX)
            nk = k + NBUF
            if nk < NCHUNK:
                _wait_idx(nk, nk % NIDX)
                _gather(nk % NBUF, nk % NIDX)
        plsc.subcore_barrier()
        pltpu.sync_copy(acc_sh.at[pl.ds(s * RT, RT)],
                        out_hbm.at[c, pl.ds(s * RT, RT)])

    return _agg_kernel


_agg128 = _make_agg(DH)


_TR = 256


def _dinv_col(degp_ref):
    dsum = degp_ref[0, :] + degp_ref[1, :] + 1.0
    dl = lax.rsqrt(dsum)
    rows = lax.broadcasted_iota(jnp.int32, (_TR, _TR), 0)
    cols = lax.broadcasted_iota(jnp.int32, (_TR, _TR), 1)
    eye = (rows == cols)
    cols_out = []
    for t in range(R // _TR):
        diag = jnp.where(eye, dl[None, t * _TR:(t + 1) * _TR], 0.0)
        cols_out.append(jnp.sum(diag, axis=1, keepdims=True))
    return jnp.concatenate(cols_out, axis=0)


def _mm1_body(x_ref, w_ref, h_ref):
    h_ref[...] = jnp.dot(x_ref[...], w_ref[...],
                         preferred_element_type=jnp.float32)


def _scale_body(h_ref, degp_ref, hs_ref, dinv_ref):
    col = _dinv_col(degp_ref)
    hs_ref[...] = h_ref[...] * col
    dinv_ref[...] = col


def _mm2_body(aggp_ref, h1_ref, dinv_ref, b1_ref, w2_ref, h2_ref):
    agg = aggp_ref[0] + aggp_ref[1]
    z = dinv_ref[...] * (agg + h1_ref[...]) + b1_ref[...]
    z = jnp.maximum(z, 0.0)
    h2 = jnp.dot(z, w2_ref[...], preferred_element_type=jnp.float32)
    h2_ref[...] = jnp.concatenate(
        [h2 * dinv_ref[...], jnp.zeros((R, DH - DC), jnp.float32)], axis=1)


def _out_body(aggp_ref, h2_ref, dinv_ref, b2_ref, out_ref):
    acc = aggp_ref[0] + aggp_ref[1] + h2_ref[...]
    z = dinv_ref[...] * acc[:, :DC] + b2_ref[...]
    m = jnp.max(z, axis=1, keepdims=True)
    lse = jnp.log(jnp.sum(jnp.exp(z - m), axis=1, keepdims=True))
    out_ref[...] = z - m - lse


def _split_body(e_ref, src_ref, dst_ref):
    src_ref[...] = e_ref[0, :]
    dst_ref[...] = e_ref[1, :]


def _tc_split(edges):
    return pl.pallas_call(
        _split_body,
        out_shape=[
            jax.ShapeDtypeStruct((E,), jnp.int32),
            jax.ShapeDtypeStruct((E,), jnp.int32),
        ],
    )(edges)


def kernel(features, edges, W1, b1, W2, b2):
    src, dst = _tc_split(edges.astype(jnp.int32))

    degp = _deg_kernel(dst)
    h1 = _tc_mm1(features, W1)
    h1p, dinvc = _tc_scale(h1, degp)
    zrow = jnp.zeros((RT, DH), jnp.float32)
    agg1 = _agg128(h1p, src, dst, zrow)
    h2p = _tc_layer2(agg1, h1p, dinvc, b1.reshape(1, DH), W2)
    agg2 = _agg128(h2p, src, dst, zrow)
    return _tc_out(agg2, h2p, dinvc, b2.reshape(1, DC))


def _tc_mm1(x, W1):
    return pl.pallas_call(
        _mm1_body,
        grid=(GRID,),
        in_specs=[
            pl.BlockSpec((R, DF), lambda i: (i, 0)),
            pl.BlockSpec((DF, DH), lambda i: (0, 0)),
        ],
        out_specs=pl.BlockSpec((R, DH), lambda i: (i, 0)),
        out_shape=jax.ShapeDtypeStruct((NPAD, DH), jnp.float32),
    )(x, W1)


def _tc_scale(h1, degp):
    return pl.pallas_call(
        _scale_body,
        grid=(GRID,),
        in_specs=[
            pl.BlockSpec((R, DH), lambda i: (i, 0)),
            pl.BlockSpec((NC, R), lambda i: (0, i)),
        ],
        out_specs=[
            pl.BlockSpec((R, DH), lambda i: (i, 0)),
            pl.BlockSpec((R, 1), lambda i: (i, 0)),
        ],
        out_shape=[
            jax.ShapeDtypeStruct((NPAD, DH), jnp.float32),
            jax.ShapeDtypeStruct((NPAD, 1), jnp.float32),
        ],
    )(h1, degp)


def _tc_layer2(agg1, h1p, dinvc, b1, W2):
    return pl.pallas_call(
        _mm2_body,
        grid=(GRID,),
        in_specs=[
            pl.BlockSpec((NC, R, DH), lambda i: (0, i, 0)),
            pl.BlockSpec((R, DH), lambda i: (i, 0)),
            pl.BlockSpec((R, 1), lambda i: (i, 0)),
            pl.BlockSpec((1, DH), lambda i: (0, 0)),
            pl.BlockSpec((DH, DC), lambda i: (0, 0)),
        ],
        out_specs=pl.BlockSpec((R, DH), lambda i: (i, 0)),
        out_shape=jax.ShapeDtypeStruct((NPAD, DH), jnp.float32),
    )(agg1, h1p, dinvc, b1, W2)


RO = N // GRID


def _tc_out(agg2, h2p, dinvc, b2):
    return pl.pallas_call(
        _out_body,
        grid=(GRID,),
        in_specs=[
            pl.BlockSpec((NC, RO, DH), lambda i: (0, i, 0)),
            pl.BlockSpec((RO, DH), lambda i: (i, 0)),
            pl.BlockSpec((RO, 1), lambda i: (i, 0)),
            pl.BlockSpec((1, DC), lambda i: (0, 0)),
        ],
        out_specs=pl.BlockSpec((RO, DC), lambda i: (i, 0)),
        out_shape=jax.ShapeDtypeStruct((N, DC), jnp.float32),
    )(agg2, h2p, dinvc, b2)
